# baseline (speedup 1.0000x reference)
import jax
import jax.numpy as jnp
import numpy as np
from jax import lax
from jax.experimental import pallas as pl

_GRID = 32
_NSIDE = 2 * _GRID + 1
_NCLS = _NSIDE * _NSIDE
_B = 4096
_EPS = 1e-10
_ROWS = 256

_TF_PARITY = 0x1BD11BDA
_TF_ROTS = ((13, 15, 26, 6), (17, 29, 16, 24))


def _threefry_bits(p, k0, k1):
    ks = (k0, k1, k0 ^ k1 ^ np.uint32(_TF_PARITY))
    x0 = jnp.zeros_like(p) + ks[0]
    x1 = p + ks[1]
    for i in range(5):
        for r in _TF_ROTS[i % 2]:
            x0 = x0 + x1
            x1 = (x1 << r) | (x1 >> (32 - r))
            x1 = x1 ^ x0
        x0 = x0 + ks[(i + 1) % 3]
        x1 = x1 + ks[(i + 2) % 3] + np.uint32(i + 1)
    return x0 ^ x1


def _bits_to_unit_float(bits):
    fb = (bits >> 9) | np.uint32(0x3F800000)
    return lax.bitcast_convert_type(fb, jnp.float32) - np.float32(1.0)


def _make_tc_body(k0_int, k1_int, row0=0):
    k0 = np.uint32(k0_int)
    k1 = np.uint32(k1_int)

    def body(s_ref, g_ref, y_ref, sg_ref, sm_ref, fp_ref):
        s = s_ref[...]
        g = g_ref[...]
        rows = s.shape[0]
        lane = lax.broadcasted_iota(jnp.int32, (rows, _NCLS), 1)
        valid = lane < _NCLS

        gl = s + g
        m = jnp.max(jnp.where(valid, gl, -jnp.inf), axis=1, keepdims=True)
        e = jnp.exp(gl - m)
        denom = jnp.sum(jnp.where(valid, e, 0.0), axis=1, keepdims=True)
        sg = jnp.maximum(e / denom, _EPS)

        vmax = jnp.max(jnp.where(valid, sg, -jnp.inf), axis=1, keepdims=True)
        cand = jnp.where((sg == vmax) & valid, lane, jnp.int32(_NCLS))
        idx = jnp.min(cand, axis=1, keepdims=True)

        hard = (lane == idx).astype(jnp.float32)
        y_ref[...] = (hard - sg) + sg
        sg_ref[...] = sg

        m2 = jnp.max(jnp.where(valid, s, -jnp.inf), axis=1, keepdims=True)
        e2 = jnp.exp(s - m2)
        sm_ref[...] = e2 / jnp.sum(jnp.where(valid, e2, 0.0), axis=1, keepdims=True)

        fi = idx.astype(jnp.float32)
        col_div = jnp.floor((fi + 0.5) * np.float32(1.0 / _NSIDE))
        base0 = fi - col_div * _NSIDE - _GRID
        base1 = col_div - _GRID
        row = (lax.broadcasted_iota(jnp.int32, (rows, 1), 0)
               + pl.program_id(0) * rows + np.int32(row0))
        flat = row * _NCLS + idx
        p0 = flat.astype(jnp.uint32) * np.uint32(2)
        j0 = _bits_to_unit_float(_threefry_bits(p0, k0, k1))
        j1 = _bits_to_unit_float(_threefry_bits(p0 + np.uint32(1), k0, k1))
        yi = (1.0 - vmax) + vmax
        fp_ref[...] = jnp.concatenate(
            [(base0 + j0) * yi, (base1 + j1) * yi], axis=1)

    return body


_CACHE = {}


def _consts():
    if "c" not in _CACHE:
        with jax.ensure_compile_time_eval():
            k1, k2 = jax.random.split(jax.random.key(1))
            u = jax.random.uniform(k2, (_B, _NCLS), dtype=jnp.float32)
            gumbel = -jnp.log(_EPS - jnp.log(u + _EPS))
            kd = jax.random.key_data(k1)
            _CACHE["c"] = (gumbel, int(kd[0]), int(kd[1]))
    return _CACHE["c"]


_CHUNKS = 2


def kernel(cnn_out):
    b, c, hh, w = cnn_out.shape
    n = hh * w
    gumbel, k0, k1 = _consts()

    ch = b // _CHUNKS
    nb = ch // _ROWS
    spec = pl.BlockSpec((_ROWS, n), lambda i: (i, 0))
    parts = []
    for k in range(_CHUNKS):
        scores_k = cnn_out[k * ch:(k + 1) * ch].reshape(ch, n)
        parts.append((scores_k,) + pl.pallas_call(
            _make_tc_body(k0, k1, row0=k * ch),
            grid=(nb,),
            in_specs=[spec, spec],
            out_specs=[spec, spec, spec,
                       pl.BlockSpec((_ROWS, 2), lambda i: (i, 0))],
            out_shape=[
                jax.ShapeDtypeStruct((ch, n), jnp.float32),
                jax.ShapeDtypeStruct((ch, n), jnp.float32),
                jax.ShapeDtypeStruct((ch, n), jnp.float32),
                jax.ShapeDtypeStruct((ch, 2), jnp.float32),
            ],
        )(scores_k, lax.slice_in_dim(gumbel, k * ch, (k + 1) * ch)))

    def cat4(i):
        return jnp.concatenate(
            [p[i].reshape(ch, c, hh, w) for p in parts], axis=0)

    fp = jnp.concatenate([p[4] for p in parts], axis=0)
    scores = jnp.concatenate([p[0] for p in parts], axis=0)
    return (fp[None], cat4(1), cat4(2), cat4(3), scores)


# speedup vs baseline: 1.3077x; 1.3077x over previous
import jax
import jax.numpy as jnp
import numpy as np
from jax import lax
from jax.experimental import pallas as pl

_GRID = 32
_NSIDE = 2 * _GRID + 1
_NCLS = _NSIDE * _NSIDE
_B = 4096
_EPS = 1e-10
_ROWS = 256

_TF_PARITY = 0x1BD11BDA
_TF_ROTS = ((13, 15, 26, 6), (17, 29, 16, 24))


def _threefry_bits(p, k0, k1):
    ks = (k0, k1, k0 ^ k1 ^ np.uint32(_TF_PARITY))
    x0 = jnp.zeros_like(p) + ks[0]
    x1 = p + ks[1]
    for i in range(5):
        for r in _TF_ROTS[i % 2]:
            x0 = x0 + x1
            x1 = (x1 << r) | (x1 >> (32 - r))
            x1 = x1 ^ x0
        x0 = x0 + ks[(i + 1) % 3]
        x1 = x1 + ks[(i + 2) % 3] + np.uint32(i + 1)
    return x0 ^ x1


def _bits_to_unit_float(bits):
    fb = (bits >> 9) | np.uint32(0x3F800000)
    return lax.bitcast_convert_type(fb, jnp.float32) - np.float32(1.0)


def _make_tc_body(k0_int, k1_int, row0=0):
    k0 = np.uint32(k0_int)
    k1 = np.uint32(k1_int)

    def body(s_ref, g_ref, y_ref, sg_ref, sm_ref, fp_ref):
        s = s_ref[...]
        g = g_ref[...]
        rows = s.shape[0]
        lane = lax.broadcasted_iota(jnp.int32, (rows, _NCLS), 1)
        valid = lane < _NCLS

        gl = s + g
        m = jnp.max(jnp.where(valid, gl, -jnp.inf), axis=1, keepdims=True)
        e = jnp.exp(gl - m)
        denom = jnp.sum(jnp.where(valid, e, 0.0), axis=1, keepdims=True)
        sg = jnp.maximum(e / denom, _EPS)

        vmax = jnp.max(jnp.where(valid, sg, -jnp.inf), axis=1, keepdims=True)
        cand = jnp.where((sg == vmax) & valid, lane, jnp.int32(_NCLS))
        idx = jnp.min(cand, axis=1, keepdims=True)

        hard = (lane == idx).astype(jnp.float32)
        y_ref[...] = (hard - sg) + sg
        sg_ref[...] = sg

        m2 = jnp.max(jnp.where(valid, s, -jnp.inf), axis=1, keepdims=True)
        e2 = jnp.exp(s - m2)
        sm_ref[...] = e2 / jnp.sum(jnp.where(valid, e2, 0.0), axis=1, keepdims=True)

        fi = idx.astype(jnp.float32)
        col_div = jnp.floor((fi + 0.5) * np.float32(1.0 / _NSIDE))
        base0 = fi - col_div * _NSIDE - _GRID
        base1 = col_div - _GRID
        row = (lax.broadcasted_iota(jnp.int32, (rows, 1), 0)
               + pl.program_id(0) * rows + np.int32(row0))
        flat = row * _NCLS + idx
        p0 = flat.astype(jnp.uint32) * np.uint32(2)
        j0 = _bits_to_unit_float(_threefry_bits(p0, k0, k1))
        j1 = _bits_to_unit_float(_threefry_bits(p0 + np.uint32(1), k0, k1))
        yi = (1.0 - vmax) + vmax
        fp_ref[...] = jnp.concatenate(
            [(base0 + j0) * yi, (base1 + j1) * yi], axis=1)

    return body


_CACHE = {}


def _consts():
    if "c" not in _CACHE:
        with jax.ensure_compile_time_eval():
            k1, k2 = jax.random.split(jax.random.key(1))
            u = jax.random.uniform(k2, (_B, _NCLS), dtype=jnp.float32)
            gumbel = -jnp.log(_EPS - jnp.log(u + _EPS))
            kd = jax.random.key_data(k1)
            _CACHE["c"] = (gumbel, int(kd[0]), int(kd[1]))
    return _CACHE["c"]


def kernel(cnn_out):
    b, c, hh, w = cnn_out.shape
    n = hh * w
    gumbel, k0, k1 = _consts()
    scores = cnn_out.reshape(b, n)

    nb = b // _ROWS
    spec = pl.BlockSpec((_ROWS, n), lambda i: (i, 0))
    y, sg, sm, fp = pl.pallas_call(
        _make_tc_body(k0, k1),
        grid=(nb,),
        in_specs=[spec, spec],
        out_specs=[spec, spec, spec,
                   pl.BlockSpec((_ROWS, 2), lambda i: (i, 0))],
        out_shape=[
            jax.ShapeDtypeStruct((b, n), jnp.float32),
            jax.ShapeDtypeStruct((b, n), jnp.float32),
            jax.ShapeDtypeStruct((b, n), jnp.float32),
            jax.ShapeDtypeStruct((b, 2), jnp.float32),
        ],
    )(scores, gumbel)

    return (
        fp[None],
        y.reshape(b, c, hh, w),
        sg.reshape(b, c, hh, w),
        sm.reshape(b, c, hh, w),
        scores,
    )
